# Initial kernel scaffold; baseline (speedup 1.0000x reference)
import jax
import jax.numpy as jnp
from jax.experimental import pallas as pl


def kernel(edge_index, embed, W_rel, b_rel, W_root, W_anti, anti_bias, W_lin, b_lin):
    raise NotImplementedError("write your pallas kernel here")


# speedup vs baseline: 8.9423x; 8.9423x over previous
import functools

import jax
import jax.numpy as jnp
from jax import lax
from jax.experimental import pallas as pl
from jax.experimental.pallas import tpu as pltpu
from jax.experimental.pallas import tpu_sc as plsc

_N = 10000
_C = 128
_E = 320000
_O = 40
_GAMMA = 0.1
_EPS = 0.1

_NC = 2
_NS = 16
_NW = _NC * _NS
_EPW = _E // _NW
_K = 100
_PHASES = 2
_PCH = _EPW // (_PHASES * _K)
_RPT = 624
_RTAIL = _N - _NS * _RPT

_mesh = plsc.VectorSubcoreMesh(
    core_axis_name="c", subcore_axis_name="s", num_cores=_NC, num_subcores=_NS
)


@functools.partial(
    pl.kernel,
    out_type=jax.ShapeDtypeStruct((_NC, _N, _C), jnp.float32),
    mesh=_mesh,
    scratch_types=[
        pltpu.VMEM((_PCH, _K), jnp.int32),
        pltpu.VMEM((_PCH, _K), jnp.int32),
        pltpu.VMEM((_K, _C), jnp.float32),
        pltpu.VMEM((_K, _C), jnp.float32),
        pltpu.VMEM_SHARED((_N, _C), jnp.float32),
        pltpu.SemaphoreType.DMA,
    ],
)
def _sc_aggregate(src_hbm, dst_hbm, x_hbm, z_hbm, out_hbm,
                  src_v, dst_v, rows_a, rows_b, acc, sem):
    c = lax.axis_index("c")
    s = lax.axis_index("s")
    wid = c * _NS + s

    pltpu.sync_copy(z_hbm.at[pl.ds(s * _RPT, _RPT)],
                    acc.at[pl.ds(s * _RPT, _RPT)])

    @pl.when(s == _NS - 1)
    def _():
        pltpu.sync_copy(z_hbm.at[pl.ds(_NS * _RPT, _RTAIL)],
                        acc.at[pl.ds(_NS * _RPT, _RTAIL)])
    plsc.subcore_barrier()

    def phase(ph, carry):
        pltpu.sync_copy(src_hbm.at[wid, ph], src_v)
        pltpu.sync_copy(dst_hbm.at[wid, ph], dst_v)

        pltpu.async_copy(x_hbm.at[src_v.at[0]], rows_a, sem)

        def pair(p, carry2):
            j = 2 * p
            pltpu.make_async_copy(x_hbm.at[src_v.at[j]], rows_a, sem).wait()
            pltpu.async_copy(x_hbm.at[src_v.at[j + 1]], rows_b, sem)
            pltpu.sync_copy(rows_a, acc.at[dst_v.at[j]], add=True)
            pltpu.make_async_copy(x_hbm.at[src_v.at[j + 1]], rows_b, sem).wait()

            @pl.when(j + 2 < _PCH)
            def _():
                pltpu.async_copy(x_hbm.at[src_v.at[j + 2]], rows_a, sem)

            pltpu.sync_copy(rows_b, acc.at[dst_v.at[j + 1]], add=True)
            return carry2

        lax.fori_loop(0, _PCH // 2, pair, 0)
        return carry

    lax.fori_loop(0, _PHASES, phase, 0)
    plsc.subcore_barrier()

    pltpu.sync_copy(acc.at[pl.ds(s * _RPT, _RPT)],
                    out_hbm.at[c, pl.ds(s * _RPT, _RPT)])

    @pl.when(s == _NS - 1)
    def _():
        pltpu.sync_copy(acc.at[pl.ds(_NS * _RPT, _RTAIL)],
                        out_hbm.at[c, pl.ds(_NS * _RPT, _RTAIL)])


_ROWS_BLK = 1000


def _tc_body(x_ref, p_ref, w1_ref, m_ref, bias_ref, w3_ref, b3_ref, o_ref):
    x = x_ref[...]
    agg = p_ref[0] + p_ref[1]
    h = jnp.tanh(
        jnp.dot(agg, w1_ref[...], preferred_element_type=jnp.float32)
        + jnp.dot(x, m_ref[...], preferred_element_type=jnp.float32)
        + bias_ref[...]
    )
    xn = x + _EPS * h
    o_ref[...] = jax.nn.sigmoid(
        jnp.dot(xn, w3_ref[...], preferred_element_type=jnp.float32)
        + b3_ref[...]
    )


def _tc_dense(x, partials, w1, m, bias, w3, b3):
    grid = (_N // _ROWS_BLK,)
    return pl.pallas_call(
        _tc_body,
        grid=grid,
        in_specs=[
            pl.BlockSpec((_ROWS_BLK, _C), lambda i: (i, 0)),
            pl.BlockSpec((_NC, _ROWS_BLK, _C), lambda i: (0, i, 0)),
            pl.BlockSpec((_C, _C), lambda i: (0, 0)),
            pl.BlockSpec((_C, _C), lambda i: (0, 0)),
            pl.BlockSpec((1, _C), lambda i: (0, 0)),
            pl.BlockSpec((_C, _O), lambda i: (0, 0)),
            pl.BlockSpec((1, _O), lambda i: (0, 0)),
        ],
        out_specs=pl.BlockSpec((_ROWS_BLK, _O), lambda i: (i, 0)),
        out_shape=jax.ShapeDtypeStruct((_N, _O), jnp.float32),
    )(x, partials, w1, m, bias, w3, b3)


def kernel(edge_index, embed, W_rel, b_rel, W_root, W_anti, anti_bias,
           W_lin, b_lin):
    src = edge_index[0].astype(jnp.int32).reshape(_NW, _PHASES, _PCH, _K)
    dst = edge_index[1].astype(jnp.int32).reshape(_NW, _PHASES, _PCH, _K)
    zeros = jnp.zeros((_N, _C), jnp.float32)
    partials = _sc_aggregate(src, dst, embed, zeros)

    m = W_anti.T - W_anti - _GAMMA * jnp.eye(_C, dtype=jnp.float32) + W_root.T
    bias = (b_rel + anti_bias).reshape(1, _C)
    return _tc_dense(embed, partials, W_rel.T, m, bias, W_lin.T,
                     b_lin.reshape(1, _O))


# speedup vs baseline: 9.9696x; 1.1149x over previous
import functools

import jax
import jax.numpy as jnp
from jax import lax
from jax.experimental import pallas as pl
from jax.experimental.pallas import tpu as pltpu
from jax.experimental.pallas import tpu_sc as plsc

_N = 10000
_C = 128
_E = 320000
_O = 40
_GAMMA = 0.1
_EPS = 0.1

_NC = 2
_NS = 16
_NW = _NC * _NS
_EPW = _E // _NW
_K = 125
_PHASES = 2
_PCH = _EPW // (_PHASES * _K)
_RPT = 624
_RTAIL = _N - _NS * _RPT

_mesh = plsc.VectorSubcoreMesh(
    core_axis_name="c", subcore_axis_name="s", num_cores=_NC, num_subcores=_NS
)


@functools.partial(
    pl.kernel,
    out_type=jax.ShapeDtypeStruct((_NC, _N, _C), jnp.float32),
    mesh=_mesh,
    scratch_types=[
        pltpu.VMEM((_PCH, _K), jnp.int32),
        pltpu.VMEM((_PCH, _K), jnp.int32),
        pltpu.VMEM((_K, _C), jnp.float32),
        pltpu.VMEM((_K, _C), jnp.float32),
        pltpu.VMEM_SHARED((_N, _C), jnp.float32),
        pltpu.SemaphoreType.DMA,
    ],
)
def _sc_aggregate(edges_hbm, x_hbm, z_hbm, out_hbm,
                  src_v, dst_v, rows_a, rows_b, acc, sem):
    c = lax.axis_index("c")
    s = lax.axis_index("s")
    wid = c * _NS + s

    pltpu.sync_copy(z_hbm.at[pl.ds(s * _RPT, _RPT)],
                    acc.at[pl.ds(s * _RPT, _RPT)])

    @pl.when(s == _NS - 1)
    def _():
        pltpu.sync_copy(z_hbm.at[pl.ds(_NS * _RPT, _RTAIL)],
                        acc.at[pl.ds(_NS * _RPT, _RTAIL)])
    plsc.subcore_barrier()

    def phase(ph, carry):
        pltpu.sync_copy(edges_hbm.at[0, wid, ph], src_v)
        pltpu.sync_copy(edges_hbm.at[1, wid, ph], dst_v)

        pltpu.async_copy(x_hbm.at[src_v.at[0]], rows_a, sem)

        def pair(p, carry2):
            j = 2 * p
            pltpu.make_async_copy(x_hbm.at[src_v.at[j]], rows_a, sem).wait()
            pltpu.async_copy(x_hbm.at[src_v.at[j + 1]], rows_b, sem)
            pltpu.sync_copy(rows_a, acc.at[dst_v.at[j]], add=True)
            pltpu.make_async_copy(x_hbm.at[src_v.at[j + 1]], rows_b, sem).wait()

            @pl.when(j + 2 < _PCH)
            def _():
                pltpu.async_copy(x_hbm.at[src_v.at[j + 2]], rows_a, sem)

            pltpu.sync_copy(rows_b, acc.at[dst_v.at[j + 1]], add=True)
            return carry2

        lax.fori_loop(0, _PCH // 2, pair, 0)
        return carry

    lax.fori_loop(0, _PHASES, phase, 0)
    plsc.subcore_barrier()

    pltpu.sync_copy(acc.at[pl.ds(s * _RPT, _RPT)],
                    out_hbm.at[c, pl.ds(s * _RPT, _RPT)])

    @pl.when(s == _NS - 1)
    def _():
        pltpu.sync_copy(acc.at[pl.ds(_NS * _RPT, _RTAIL)],
                        out_hbm.at[c, pl.ds(_NS * _RPT, _RTAIL)])


_ROWS_BLK = 1000


def _tc_body(x_ref, p_ref, w1_ref, m_ref, bias_ref, w3_ref, b3_ref, o_ref):
    x = x_ref[...]
    agg = p_ref[0] + p_ref[1]
    h = jnp.tanh(
        jnp.dot(agg, w1_ref[...], preferred_element_type=jnp.float32)
        + jnp.dot(x, m_ref[...], preferred_element_type=jnp.float32)
        + bias_ref[...]
    )
    xn = x + _EPS * h
    o_ref[...] = jax.nn.sigmoid(
        jnp.dot(xn, w3_ref[...], preferred_element_type=jnp.float32)
        + b3_ref[...]
    )


def _tc_dense(x, partials, w1, m, bias, w3, b3):
    grid = (_N // _ROWS_BLK,)
    return pl.pallas_call(
        _tc_body,
        grid=grid,
        in_specs=[
            pl.BlockSpec((_ROWS_BLK, _C), lambda i: (i, 0)),
            pl.BlockSpec((_NC, _ROWS_BLK, _C), lambda i: (0, i, 0)),
            pl.BlockSpec((_C, _C), lambda i: (0, 0)),
            pl.BlockSpec((_C, _C), lambda i: (0, 0)),
            pl.BlockSpec((1, _C), lambda i: (0, 0)),
            pl.BlockSpec((_C, _O), lambda i: (0, 0)),
            pl.BlockSpec((1, _O), lambda i: (0, 0)),
        ],
        out_specs=pl.BlockSpec((_ROWS_BLK, _O), lambda i: (i, 0)),
        out_shape=jax.ShapeDtypeStruct((_N, _O), jnp.float32),
    )(x, partials, w1, m, bias, w3, b3)


def kernel(edge_index, embed, W_rel, b_rel, W_root, W_anti, anti_bias,
           W_lin, b_lin):
    edges = edge_index.astype(jnp.int32).reshape(2, _NW, _PHASES, _PCH, _K)
    zeros = jnp.zeros((_N, _C), jnp.float32)
    partials = _sc_aggregate(edges, embed, zeros)

    m = W_anti.T - W_anti - _GAMMA * jnp.eye(_C, dtype=jnp.float32) + W_root.T
    bias = (b_rel + anti_bias).reshape(1, _C)
    return _tc_dense(embed, partials, W_rel.T, m, bias, W_lin.T,
                     b_lin.reshape(1, _O))
